# Initial kernel scaffold; baseline (speedup 1.0000x reference)
import functools

import jax
import jax.numpy as jnp
from jax import lax
from jax.experimental import pallas as pl
from jax.experimental.pallas import tpu as pltpu
from jax.experimental.pallas import tpu_sc as plsc

_D = 64
_S = 200
_NC = 2
_NS = 16
_NW = _NC * _NS
_G0 = 128
_G1 = _S - _G0


def _tpe_sc(xf, emb, pos):
    n = xf.shape[0]
    seq_per_tile = (n // _S) // _NW
    mesh = plsc.VectorSubcoreMesh(core_axis_name="c", subcore_axis_name="s")

    @functools.partial(
        pl.kernel,
        mesh=mesh,
        out_type=jax.ShapeDtypeStruct((n, _D), jnp.float32),
        scratch_types=[
            pltpu.VMEM((_S, _D), jnp.float32),
            pltpu.VMEM((_G0,), jnp.int32),
            pltpu.VMEM((_G1,), jnp.int32),
            pltpu.VMEM((_S, _D), jnp.float32),
            pltpu.SemaphoreType.DMA,
        ],
    )
    def k(emb_hbm, idx_hbm, pos_hbm, out_hbm, pos_v, idx_a, idx_b, rows_v, sem):
        wid = lax.axis_index("s") * _NC + lax.axis_index("c")
        pltpu.sync_copy(pos_hbm, pos_v)

        @pl.loop(0, seq_per_tile)
        def _(c):
            row0 = (wid * seq_per_tile + c) * _S
            pltpu.sync_copy(idx_hbm.at[pl.ds(row0, _G0)], idx_a)
            pltpu.sync_copy(idx_hbm.at[pl.ds(row0 + _G0, _G1)], idx_b)
            pltpu.async_copy(emb_hbm.at[idx_a], rows_v.at[pl.ds(0, _G0)],
                             sem).wait()
            pltpu.async_copy(emb_hbm.at[idx_b], rows_v.at[pl.ds(_G0, _G1)],
                             sem).wait()

            @pl.loop(0, _S)
            def _(j):
                for d0 in range(0, _D, 16):
                    sl = (pl.ds(j, 1), pl.ds(d0, 16))
                    rows_v.at[sl][...] = rows_v.at[sl][...] + pos_v.at[sl][...]

            pltpu.sync_copy(rows_v, out_hbm.at[pl.ds(row0, _S)])

    return k(emb, xf, pos)


def kernel(x, embedding, pos_embedding):
    b, s = x.shape
    xf = x.reshape(-1).astype(jnp.int32)
    out = _tpe_sc(xf, embedding, pos_embedding)
    return out.reshape(b, s, _D)


# speedup vs baseline: 2.3648x; 2.3648x over previous
import functools

import jax
import jax.numpy as jnp
from jax import lax
from jax.experimental import pallas as pl
from jax.experimental.pallas import tpu as pltpu
from jax.experimental.pallas import tpu_sc as plsc

_D = 64
_S = 200
_NC = 2
_NS = 16
_NW = _NC * _NS
_G0 = 128
_G1 = _S - _G0


def _tpe_sc(xf, emb, pos):
    n = xf.shape[0]
    seq_per_tile = (n // _S) // _NW
    mesh = plsc.VectorSubcoreMesh(core_axis_name="c", subcore_axis_name="s")

    @functools.partial(
        pl.kernel,
        mesh=mesh,
        compiler_params=pltpu.CompilerParams(use_tc_tiling_on_sc=False),
        out_type=jax.ShapeDtypeStruct((n, _D), jnp.float32),
        scratch_types=[
            pltpu.VMEM((_S, _D), jnp.float32),
            pltpu.VMEM((_G0,), jnp.int32),
            pltpu.VMEM((_G1,), jnp.int32),
            pltpu.VMEM((_S, _D), jnp.float32),
            pltpu.SemaphoreType.DMA,
        ],
    )
    def k(emb_hbm, idx_hbm, pos_hbm, out_hbm, pos_v, idx_a, idx_b, rows_v, sem):
        wid = lax.axis_index("s") * _NC + lax.axis_index("c")
        pltpu.sync_copy(pos_hbm, pos_v)

        @pl.loop(0, seq_per_tile)
        def _(c):
            row0 = (wid * seq_per_tile + c) * _S
            pltpu.sync_copy(idx_hbm.at[pl.ds(row0, _G0)], idx_a)
            pltpu.sync_copy(idx_hbm.at[pl.ds(row0 + _G0, _G1)], idx_b)
            pltpu.async_copy(emb_hbm.at[idx_a], rows_v.at[pl.ds(0, _G0)],
                             sem).wait()
            pltpu.async_copy(emb_hbm.at[idx_b], rows_v.at[pl.ds(_G0, _G1)],
                             sem).wait()

            @pl.loop(0, _S)
            def _(j):
                for d0 in range(0, _D, 16):
                    sl = (pl.ds(j, 1), pl.ds(d0, 16))
                    rows_v.at[sl][...] = rows_v.at[sl][...] + pos_v.at[sl][...]

            pltpu.sync_copy(rows_v, out_hbm.at[pl.ds(row0, _S)])

    return k(emb, xf, pos)


def kernel(x, embedding, pos_embedding):
    b, s = x.shape
    xf = x.reshape(-1).astype(jnp.int32)
    out = _tpe_sc(xf, embedding, pos_embedding)
    return out.reshape(b, s, _D)


# speedup vs baseline: 3.1186x; 1.3188x over previous
import functools

import jax
import jax.numpy as jnp
from jax import lax
from jax.experimental import pallas as pl
from jax.experimental.pallas import tpu as pltpu
from jax.experimental.pallas import tpu_sc as plsc

_D = 64
_S = 200
_NC = 2
_NS = 16
_NW = _NC * _NS
_G0 = 128
_G1 = _S - _G0
_NB = 2


def _tpe_sc(xf, emb, pos):
    n = xf.shape[0]
    spt = (n // _S) // _NW
    mesh = plsc.VectorSubcoreMesh(core_axis_name="c", subcore_axis_name="s")

    @functools.partial(
        pl.kernel,
        mesh=mesh,
        compiler_params=pltpu.CompilerParams(use_tc_tiling_on_sc=False),
        out_type=jax.ShapeDtypeStruct((n, _D), jnp.float32),
        scratch_types=[
            pltpu.VMEM((_S, _D), jnp.float32),
            pltpu.VMEM((spt * _S,), jnp.int32),
            pltpu.VMEM((_NB, _S, _D), jnp.float32),
            pltpu.SemaphoreType.DMA((_NB,)),
            pltpu.SemaphoreType.DMA((_NB,)),
        ],
    )
    def k(emb_hbm, idx_hbm, pos_hbm, out_hbm, pos_v, idx_all, rows, gsem, osem):
        wid = lax.axis_index("s") * _NC + lax.axis_index("c")
        tbase = wid * spt * _S
        pltpu.sync_copy(idx_hbm.at[pl.ds(tbase, spt * _S)], idx_all)
        pltpu.sync_copy(pos_hbm, pos_v)

        def start_gather(nloc, j):
            off = nloc * _S
            pltpu.async_copy(emb_hbm.at[idx_all.at[pl.ds(off, _G0)]],
                             rows.at[j].at[pl.ds(0, _G0)], gsem.at[j])
            pltpu.async_copy(emb_hbm.at[idx_all.at[pl.ds(off + _G0, _G1)]],
                             rows.at[j].at[pl.ds(_G0, _G1)], gsem.at[j])

        for j in range(_NB):
            start_gather(j, j)

        @pl.loop(0, spt, step=_NB)
        def _(c):
            for j in range(_NB):
                nloc = c + j
                pltpu.make_async_copy(out_hbm.at[pl.ds(0, _S)], rows.at[j],
                                      gsem.at[j]).wait()

                @pl.loop(0, _S)
                def _(r):
                    for d0 in range(0, _D, 16):
                        sl = (pl.ds(r, 1), pl.ds(d0, 16))
                        rows.at[j].at[sl][...] = (rows.at[j].at[sl][...]
                                                  + pos_v.at[sl][...])

                pltpu.async_copy(rows.at[j],
                                 out_hbm.at[pl.ds(tbase + nloc * _S, _S)],
                                 osem.at[j])

                @pl.when(nloc + _NB < spt)
                def _():
                    pltpu.make_async_copy(rows.at[j], out_hbm.at[pl.ds(0, _S)],
                                          osem.at[j]).wait()
                    start_gather(nloc + _NB, j)

        for j in range(_NB):
            pltpu.make_async_copy(rows.at[j], out_hbm.at[pl.ds(0, _S)],
                                  osem.at[j]).wait()

    return k(emb, xf, pos)


def kernel(x, embedding, pos_embedding):
    b, s = x.shape
    xf = x.reshape(-1).astype(jnp.int32)
    out = _tpe_sc(xf, embedding, pos_embedding)
    return out.reshape(b, s, _D)
